# baseline (speedup 1.0000x reference)
import jax
import jax.numpy as jnp
from jax.experimental import pallas as pl
from jax.experimental.pallas import tpu as pltpu


def _wattn_kernel(inv_scale, x_ref, m_ref, proj_ref, hid_ref, ev_ref, o_ref):
    x = x_ref[0]
    m = m_ref[0]
    xm = x * m
    w = jax.nn.sigmoid(
        jnp.dot(xm, proj_ref[...], preferred_element_type=jnp.float32)
        * inv_scale)
    for i in range(hid_ref.shape[0]):
        w = jax.nn.sigmoid(
            jnp.dot(w, hid_ref[i], preferred_element_type=jnp.float32)
            * inv_scale)
    s = jax.nn.sigmoid(
        jnp.sum(w * ev_ref[...], axis=1, keepdims=True) * inv_scale)
    e = jnp.exp(s)
    em = e * m
    z = jnp.sum(e)
    den = jnp.sum(em)
    num = jax.lax.dot_general(em, xm, (((0,), (0,)), ((), ())),
                              preferred_element_type=jnp.float32)
    o_ref[0] = num / (den + 1e-12 * z)


def kernel(inp, mask, projector, hidden, evaluator):
    B, S, D = inp.shape
    H = projector.shape[-1]
    inv_scale = 1.0 / float(H) ** 0.5
    m3 = mask.astype(inp.dtype)[..., None]
    ev = evaluator.reshape(1, H)

    out = pl.pallas_call(
        lambda *refs: _wattn_kernel(inv_scale, *refs),
        grid=(B,),
        in_specs=[
            pl.BlockSpec((1, S, D), lambda b: (b, 0, 0)),
            pl.BlockSpec((1, S, 1), lambda b: (b, 0, 0)),
            pl.BlockSpec((D, H), lambda b: (0, 0)),
            pl.BlockSpec(hidden.shape, lambda b: (0, 0, 0)),
            pl.BlockSpec((1, H), lambda b: (0, 0)),
        ],
        out_specs=pl.BlockSpec((1, 1, D), lambda b: (b, 0, 0)),
        out_shape=jax.ShapeDtypeStruct((B, 1, D), inp.dtype),
        compiler_params=pltpu.CompilerParams(
            dimension_semantics=("parallel",)),
    )(inp, m3, projector, hidden, ev)
    return out.reshape(B, D)
